# baseline (speedup 1.0000x reference)
import functools

import jax
import jax.numpy as jnp
from jax import lax
from jax.experimental import pallas as pl
from jax.experimental.pallas import tpu as pltpu
from jax.experimental.pallas import tpu_sc as plsc

_B, _L, _F = 16384, 50, 64
_V = 1000000
_NC, _NS = 2, 16
_NW = _NC * _NS
_BPW = _B // _NW
_CB = 8
_NCH = _BPW // _CB
_NP = _NCH // 2

_WR = 4000


def _tc_widen(table):
    def body(t_ref, o_ref):
        o_ref[:, 0:_F] = t_ref[...]
        o_ref[:, _F:128] = jnp.zeros((_WR, _F), jnp.float32)

    return pl.pallas_call(
        body,
        grid=(_V // _WR,),
        compiler_params=pltpu.CompilerParams(needs_layout_passes=False),
        in_specs=[pl.BlockSpec((_WR, _F), lambda i: (i, 0))],
        out_specs=pl.BlockSpec((_WR, 128), lambda i: (i, 0)),
        out_shape=jax.ShapeDtypeStruct((_V, 128), jnp.float32),
    )(table)


def _sc_pool(x, wide):
    mesh = plsc.VectorSubcoreMesh(core_axis_name="c", subcore_axis_name="s")

    @functools.partial(
        pl.kernel,
        out_type=jax.ShapeDtypeStruct((_B, _F), jnp.float32),
        mesh=mesh,
        compiler_params=pltpu.CompilerParams(use_tc_tiling_on_sc=True),
        scratch_types=[
            pltpu.VMEM((2, _CB, _L), jnp.int32),
            pltpu.VMEM((2, _CB, _L, 128), jnp.float32),
            pltpu.VMEM((_CB, _F), jnp.float32),
            pltpu.SemaphoreType.DMA,
            pltpu.SemaphoreType.DMA,
        ],
    )
    def k(x_hbm, wide_hbm, out_hbm, idx_v, rows_v, acc_v, sem0, sem1):
        wid = lax.axis_index("s") * _NC + lax.axis_index("c")
        base = wid * _BPW
        sems = (sem0, sem1)

        def fire(c, buf):
            bb = base + c * _CB
            pltpu.sync_copy(x_hbm.at[pl.ds(bb, _CB)], idx_v.at[buf])
            for j in range(_CB):
                pltpu.async_copy(
                    wide_hbm.at[idx_v.at[buf, j]], rows_v.at[buf, j], sems[buf]
                )

        def drain(buf):
            for j in range(_CB):
                pltpu.make_async_copy(
                    wide_hbm.at[idx_v.at[buf, j]], rows_v.at[buf, j], sems[buf]
                ).wait()

        def accum_store(c, buf):
            for j in range(_CB):
                def lbody(l, a):
                    return (
                        a[0] + rows_v[buf, j, l, pl.ds(0, 16)],
                        a[1] + rows_v[buf, j, l, pl.ds(16, 16)],
                        a[2] + rows_v[buf, j, l, pl.ds(32, 16)],
                        a[3] + rows_v[buf, j, l, pl.ds(48, 16)],
                    )

                z = jnp.zeros((16,), jnp.float32)
                a = lax.fori_loop(0, _L, lbody, (z, z, z, z))
                acc_v[j, pl.ds(0, 16)] = a[0]
                acc_v[j, pl.ds(16, 16)] = a[1]
                acc_v[j, pl.ds(32, 16)] = a[2]
                acc_v[j, pl.ds(48, 16)] = a[3]
            pltpu.sync_copy(acc_v, out_hbm.at[pl.ds(base + c * _CB, _CB)])

        fire(0, 0)

        def body(p, carry):
            c0 = 2 * p
            fire(c0 + 1, 1)
            drain(0)
            accum_store(c0, 0)

            @pl.when(c0 + 2 < _NCH)
            def _():
                fire(c0 + 2, 0)

            drain(1)
            accum_store(c0 + 1, 1)
            return carry

        lax.fori_loop(0, _NP, body, 0)

    return k(x, wide)


def _tc_post(pooled_sum, Wp, prm):
    BT = 2048

    def body(p_ref, w_ref, prm_ref, o_ref):
        h = jnp.dot(p_ref[...], w_ref[...], preferred_element_type=jnp.float32)
        h = jnp.maximum(h + prm_ref[0:1, :], 0.0)
        h = h * prm_ref[1:2, :] + prm_ref[2:3, :]
        mu = jnp.mean(h, axis=-1, keepdims=True)
        d = h - mu
        var = jnp.mean(d * d, axis=-1, keepdims=True)
        o_ref[...] = d * lax.rsqrt(var + 1e-3) * prm_ref[3:4, :] + prm_ref[4:5, :]

    return pl.pallas_call(
        body,
        grid=(_B // BT,),
        in_specs=[
            pl.BlockSpec((BT, _F), lambda i: (i, 0)),
            pl.BlockSpec((_F, _F), lambda i: (0, 0)),
            pl.BlockSpec((8, _F), lambda i: (0, 0)),
        ],
        out_specs=pl.BlockSpec((BT, _F), lambda i: (i, 0)),
        out_shape=jax.ShapeDtypeStruct((_B, _F), jnp.float32),
    )(pooled_sum, Wp, prm)


def kernel(x, table, W, b, bn_gamma, bn_beta, bn_mean, bn_var, ln_gamma, ln_beta):
    x = x.astype(jnp.int32)
    wide = _tc_widen(table)
    pooled_sum = _sc_pool(x, wide)
    bn_scale = bn_gamma * lax.rsqrt(bn_var + 1e-3)
    bn_shift = bn_beta - bn_mean * bn_scale
    prm = jnp.zeros((8, _F), jnp.float32)
    prm = prm.at[0].set(b).at[1].set(bn_scale).at[2].set(bn_shift)
    prm = prm.at[3].set(ln_gamma).at[4].set(ln_beta)
    Wp = W * (1.0 / _L)
    return _tc_post(pooled_sum, Wp, prm)


# speedup vs baseline: 1.1536x; 1.1536x over previous
import functools

import jax
import jax.numpy as jnp
from jax import lax
from jax.experimental import pallas as pl
from jax.experimental.pallas import tpu as pltpu
from jax.experimental.pallas import tpu_sc as plsc

_B, _L, _F = 16384, 50, 64
_NC, _NS = 2, 16
_NW = _NC * _NS
_BPW = _B // _NW
_CB = 8
_NCH = _BPW // _CB
_NP = _NCH // 2


def _sc_pool(x, table):
    mesh = plsc.VectorSubcoreMesh(core_axis_name="c", subcore_axis_name="s")

    @functools.partial(
        pl.kernel,
        out_type=jax.ShapeDtypeStruct((_B, _F), jnp.float32),
        mesh=mesh,
        compiler_params=pltpu.CompilerParams(use_tc_tiling_on_sc=False),
        scratch_types=[
            pltpu.VMEM((2, _CB, _L), jnp.int32),
            pltpu.VMEM((2, _CB, _L, _F), jnp.float32),
            pltpu.VMEM((_CB, _F), jnp.float32),
            pltpu.SemaphoreType.DMA,
            pltpu.SemaphoreType.DMA,
        ],
    )
    def k(x_hbm, table_hbm, out_hbm, idx_v, rows_v, acc_v, sem0, sem1):
        wid = lax.axis_index("s") * _NC + lax.axis_index("c")
        base = wid * _BPW
        sems = (sem0, sem1)

        def fire(c, buf):
            bb = base + c * _CB
            pltpu.sync_copy(x_hbm.at[pl.ds(bb, _CB)], idx_v.at[buf])
            for j in range(_CB):
                pltpu.async_copy(
                    table_hbm.at[idx_v.at[buf, j]], rows_v.at[buf, j], sems[buf]
                )

        def drain(buf):
            for j in range(_CB):
                pltpu.make_async_copy(
                    table_hbm.at[idx_v.at[buf, j]], rows_v.at[buf, j], sems[buf]
                ).wait()

        def accum_store(c, buf):
            for j in range(_CB):
                def lbody(l, a):
                    return (
                        a[0] + rows_v[buf, j, l, pl.ds(0, 16)],
                        a[1] + rows_v[buf, j, l, pl.ds(16, 16)],
                        a[2] + rows_v[buf, j, l, pl.ds(32, 16)],
                        a[3] + rows_v[buf, j, l, pl.ds(48, 16)],
                    )

                z = jnp.zeros((16,), jnp.float32)
                a = lax.fori_loop(0, _L, lbody, (z, z, z, z))
                acc_v[j, pl.ds(0, 16)] = a[0]
                acc_v[j, pl.ds(16, 16)] = a[1]
                acc_v[j, pl.ds(32, 16)] = a[2]
                acc_v[j, pl.ds(48, 16)] = a[3]
            pltpu.sync_copy(acc_v, out_hbm.at[pl.ds(base + c * _CB, _CB)])

        fire(0, 0)

        def body(p, carry):
            c0 = 2 * p
            fire(c0 + 1, 1)
            drain(0)
            accum_store(c0, 0)

            @pl.when(c0 + 2 < _NCH)
            def _():
                fire(c0 + 2, 0)

            drain(1)
            accum_store(c0 + 1, 1)
            return carry

        lax.fori_loop(0, _NP, body, 0)

    return k(x, table)


def _tc_post(pooled_sum, Wp, prm):
    BT = 2048

    def body(p_ref, w_ref, prm_ref, o_ref):
        h = jnp.dot(p_ref[...], w_ref[...], preferred_element_type=jnp.float32)
        h = jnp.maximum(h + prm_ref[0:1, :], 0.0)
        h = h * prm_ref[1:2, :] + prm_ref[2:3, :]
        mu = jnp.mean(h, axis=-1, keepdims=True)
        d = h - mu
        var = jnp.mean(d * d, axis=-1, keepdims=True)
        o_ref[...] = d * lax.rsqrt(var + 1e-3) * prm_ref[3:4, :] + prm_ref[4:5, :]

    return pl.pallas_call(
        body,
        grid=(_B // BT,),
        in_specs=[
            pl.BlockSpec((BT, _F), lambda i: (i, 0)),
            pl.BlockSpec((_F, _F), lambda i: (0, 0)),
            pl.BlockSpec((8, _F), lambda i: (0, 0)),
        ],
        out_specs=pl.BlockSpec((BT, _F), lambda i: (i, 0)),
        out_shape=jax.ShapeDtypeStruct((_B, _F), jnp.float32),
    )(pooled_sum, Wp, prm)


def kernel(x, table, W, b, bn_gamma, bn_beta, bn_mean, bn_var, ln_gamma, ln_beta):
    x = x.astype(jnp.int32)
    pooled_sum = _sc_pool(x, table)
    bn_scale = bn_gamma * lax.rsqrt(bn_var + 1e-3)
    bn_shift = bn_beta - bn_mean * bn_scale
    prm = jnp.zeros((8, _F), jnp.float32)
    prm = prm.at[0].set(b).at[1].set(bn_scale).at[2].set(bn_shift)
    prm = prm.at[3].set(ln_gamma).at[4].set(ln_beta)
    Wp = W * (1.0 / _L)
    return _tc_post(pooled_sum, Wp, prm)
